# baseline (speedup 1.0000x reference)
import functools

import jax
import jax.numpy as jnp
from jax import lax
from jax.experimental import pallas as pl
from jax.experimental.pallas import tpu as pltpu
from jax.experimental.pallas import tpu_sc as plsc

_Q = 65536
_B = 1024
_D = 128
_NC = 2
_NS = 16
_NW = _NC * _NS
_RPW = _Q // _NW
_SPW = (_Q - _B) // _NW
_NPW = _B // _NW

_MESH = plsc.VectorSubcoreMesh(core_axis_name="c", subcore_axis_name="s")


@functools.partial(
    pl.kernel,
    out_type=(
        jax.ShapeDtypeStruct((_Q, _D), jnp.float32),
        jax.ShapeDtypeStruct((_Q, _D), jnp.float32),
    ),
    mesh=_MESH,
    scratch_types=(
        pltpu.SemaphoreType.DMA,
        pltpu.SemaphoreType.DMA,
        pltpu.SemaphoreType.DMA,
    ),
)
def _sc_shift(new_hbm, keys_hbm, old_hbm, upd_hbm, sem0, sem1, sem2):
    wid = lax.axis_index("s") * _NC + lax.axis_index("c")
    rb = wid * _RPW
    sb = wid * _SPW
    nb = wid * _NPW
    c0 = pltpu.async_copy(
        keys_hbm.at[pl.ds(rb, _RPW)], old_hbm.at[pl.ds(rb, _RPW)], sem0)
    c1 = pltpu.async_copy(
        keys_hbm.at[pl.ds(sb, _SPW)], upd_hbm.at[pl.ds(sb + _B, _SPW)], sem1)
    c2 = pltpu.async_copy(
        new_hbm.at[pl.ds(nb, _NPW)], upd_hbm.at[pl.ds(nb, _NPW)], sem2)
    c0.wait()
    c1.wait()
    c2.wait()


def kernel(new_keys, keys):
    old, upd = _sc_shift(new_keys, keys)
    return (old, upd)


# speedup vs baseline: 36.4818x; 36.4818x over previous
import functools

import jax
import jax.numpy as jnp
from jax import lax
from jax.experimental import pallas as pl
from jax.experimental.pallas import tpu as pltpu
from jax.experimental.pallas import tpu_sc as plsc

_Q = 65536
_B = 1024
_D = 128
_NC = 2
_NS = 16
_NW = _NC * _NS
_RPW = _Q // _NW
_NPW = _B // _NW
_CH = 256
_NCH = _RPW // _CH

_MESH = plsc.VectorSubcoreMesh(core_axis_name="c", subcore_axis_name="s")


@functools.partial(
    pl.kernel,
    out_type=(
        jax.ShapeDtypeStruct((_Q, _D), jnp.float32),
        jax.ShapeDtypeStruct((_Q, _D), jnp.float32),
    ),
    mesh=_MESH,
    scratch_types=(
        pltpu.VMEM((_CH, _D), jnp.float32),
        pltpu.VMEM((_CH, _D), jnp.float32),
        pltpu.VMEM((_NPW, _D), jnp.float32),
        pltpu.SemaphoreType.DMA,
        pltpu.SemaphoreType.DMA,
    ),
)
def _sc_shift(new_hbm, keys_hbm, old_hbm, upd_hbm,
              buf0, buf1, nbuf, isem, osem):
    wid = lax.axis_index("s") * _NC + lax.axis_index("c")
    base = wid * _RPW

    nb = wid * _NPW
    pltpu.async_copy(new_hbm.at[pl.ds(nb, _NPW)], nbuf, isem).wait()
    head_w = pltpu.async_copy(nbuf, upd_hbm.at[pl.ds(nb, _NPW)], osem)

    bufs = (buf0, buf1)
    fetch = pltpu.async_copy(keys_hbm.at[pl.ds(base, _CH)], buf0, isem)
    head_w.wait()

    for ci in range(_NCH):
        buf = bufs[ci % 2]
        b = base + ci * _CH
        fetch.wait()
        if ci + 1 < _NCH:
            fetch = pltpu.async_copy(
                keys_hbm.at[pl.ds(b + _CH, _CH)], bufs[(ci + 1) % 2], isem)

        w_old = pltpu.async_copy(buf, old_hbm.at[pl.ds(b, _CH)], osem)

        @pl.when(b < _Q - _B)
        def _():
            pltpu.sync_copy(buf, upd_hbm.at[pl.ds(b + _B, _CH)])

        w_old.wait()


def kernel(new_keys, keys):
    old, upd = _sc_shift(new_keys, keys)
    return (old, upd)
